# Initial kernel scaffold; baseline (speedup 1.0000x reference)
import jax
import jax.numpy as jnp
from jax.experimental import pallas as pl


def kernel(edges, entity_embed, W_rel, W_self):
    raise NotImplementedError("write your pallas kernel here")


# speedup vs baseline: 3.6984x; 3.6984x over previous
import dataclasses

import jax
import jax.numpy as jnp
from jax import lax
from jax.experimental import pallas as pl
from jax.experimental.pallas import tpu as pltpu
from jax.experimental.pallas import tpu_sc as plsc

N = 10000
R = 8
D = 128
E = 320000
EE = 2 * E

NC = 2
NS = 16
NW = NC * NS
PER_W = EE // NW
CHUNK = 80
NCHUNK = PER_W // CHUNK

NP = 10240
ROWS_PER_SUB = NP // NS
DROWS = NP // 128

RBLK = 1000
EBLK = 1024


def _ht_body(h_ref, w_ref, out_ref):
    out_ref[...] = jnp.dot(h_ref[...], w_ref[0],
                           preferred_element_type=jnp.float32)


def _ht_table(h, w_rel):
    nb = N // RBLK
    return pl.pallas_call(
        _ht_body,
        grid=(nb, 2 * R),
        in_specs=[
            pl.BlockSpec((RBLK, D), lambda i, r: (i, 0)),
            pl.BlockSpec((1, D, D), lambda i, r: (r, 0, 0)),
        ],
        out_specs=pl.BlockSpec((RBLK, D), lambda i, r: (r * nb + i, 0)),
        out_shape=jax.ShapeDtypeStruct((2 * R * N, D), jnp.float32),
    )(h, w_rel)


def _sc_body(ht_hbm, gidx_hbm, dst_hbm, zrow_hbm,
             agg_out, deg_out,
             gbuf, dbuf, qbuf, rows, ones_rows, agg_sh, deg_sh):
    cid = lax.axis_index("c")
    sid = lax.axis_index("s")
    wid = sid * NC + cid
    sl = pl.ds(sid * ROWS_PER_SUB, ROWS_PER_SUB)
    pltpu.sync_copy(zrow_hbm, agg_sh.at[sl])
    pltpu.sync_copy(zrow_hbm.at[pl.ds(0, CHUNK)], ones_rows)

    @pl.when(sid == 0)
    def _():
        pltpu.sync_copy(zrow_hbm.at[pl.ds(0, DROWS)], deg_sh)

    plsc.subcore_barrier()

    base = wid * PER_W

    @pl.loop(0, NCHUNK)
    def _(c):
        off = pl.multiple_of(base + c * CHUNK, 8)
        pltpu.sync_copy(gidx_hbm.at[pl.ds(off, CHUNK)], gbuf)
        pltpu.sync_copy(dst_hbm.at[pl.ds(off, CHUNK)], dbuf)
        pltpu.sync_copy(ht_hbm.at[gbuf], rows)
        pltpu.sync_copy(rows, agg_sh.at[dbuf], add=True)

        ones16 = jnp.ones((16,), jnp.float32)
        zero16 = jnp.zeros((16,), jnp.float32)
        for k in range(CHUNK // 16):
            dv = dbuf[pl.ds(k * 16, 16)]
            qbuf[pl.ds(k * 16, 16)] = lax.shift_right_logical(dv, 7)
            evec = lax.iota(jnp.int32, 16) + (k * 16)
            gvec = lax.bitwise_and(dv, 127)
            plsc.store_scatter(ones_rows, [evec, gvec], ones16)
        pltpu.sync_copy(ones_rows, deg_sh.at[qbuf], add=True)
        for k in range(CHUNK // 16):
            dv = dbuf[pl.ds(k * 16, 16)]
            evec = lax.iota(jnp.int32, 16) + (k * 16)
            gvec = lax.bitwise_and(dv, 127)
            plsc.store_scatter(ones_rows, [evec, gvec], zero16)

    plsc.subcore_barrier()
    pltpu.sync_copy(agg_sh.at[sl], agg_out.at[cid, sl])

    @pl.when(sid == 0)
    def _():
        pltpu.sync_copy(deg_sh, deg_out.at[cid])


def _sc_aggregate(ht, gidx, dst, zrow):
    mesh = plsc.VectorSubcoreMesh(core_axis_name="c", subcore_axis_name="s",
                                  num_cores=NC, num_subcores=NS)
    cp = pltpu.CompilerParams()
    if "needs_layout_passes" in pltpu.CompilerParams.__dataclass_fields__:
        cp = dataclasses.replace(cp, needs_layout_passes=False)
    k = pl.kernel(
        _sc_body,
        out_type=[
            jax.ShapeDtypeStruct((NC, NP, D), jnp.float32),
            jax.ShapeDtypeStruct((NC, DROWS, D), jnp.float32),
        ],
        mesh=mesh,
        scratch_types=[
            pltpu.VMEM((CHUNK,), jnp.int32),
            pltpu.VMEM((CHUNK,), jnp.int32),
            pltpu.VMEM((CHUNK,), jnp.int32),
            pltpu.VMEM((CHUNK, D), jnp.float32),
            pltpu.VMEM((CHUNK, D), jnp.float32),
            pltpu.VMEM_SHARED((NP, D), jnp.float32),
            pltpu.VMEM_SHARED((DROWS, D), jnp.float32),
        ],
        compiler_params=cp,
    )
    return k(ht, gidx, dst, zrow)


def _epi_body(agg_ref, scale_ref, h_ref, ws_ref, out_ref):
    agg = agg_ref[0] + agg_ref[1]
    hs = jnp.dot(h_ref[...], ws_ref[...], preferred_element_type=jnp.float32)
    out_ref[...] = jnp.maximum(agg * scale_ref[...] + hs, 0.0)


def _epilogue(agg, scale_b, h_pad, w_self):
    nb = NP // EBLK
    return pl.pallas_call(
        _epi_body,
        grid=(nb,),
        in_specs=[
            pl.BlockSpec((NC, EBLK, D), lambda i: (0, i, 0)),
            pl.BlockSpec((EBLK, D), lambda i: (i, 0)),
            pl.BlockSpec((EBLK, D), lambda i: (i, 0)),
            pl.BlockSpec((D, D), lambda i: (0, 0)),
        ],
        out_specs=pl.BlockSpec((EBLK, D), lambda i: (i, 0)),
        out_shape=jax.ShapeDtypeStruct((NP, D), jnp.float32),
    )(agg, scale_b, h_pad, w_self)


def kernel(edges, entity_embed, W_rel, W_self):
    src = edges[:, 0]
    rel = edges[:, 1]
    dst = edges[:, 2]
    gidx = jnp.concatenate([rel * N + src, (rel + R) * N + dst])
    dall = jnp.concatenate([dst, src])

    ht = _ht_table(entity_embed, W_rel)
    zrow = jnp.zeros((ROWS_PER_SUB, D), jnp.float32)
    aggs, degs = _sc_aggregate(ht, gidx, dall, zrow)

    deg = (degs[0] + degs[1]).reshape(NP)
    scale = 1.0 / jnp.clip(deg, 1.0, None)
    scale_b = jnp.broadcast_to(scale[:, None], (NP, D))
    h_pad = jnp.pad(entity_embed, ((0, NP - N), (0, 0)))
    out = _epilogue(aggs, scale_b, h_pad, W_self)
    return out[:N]


# speedup vs baseline: 6.1342x; 1.6586x over previous
import dataclasses

import jax
import jax.numpy as jnp
from jax import lax
from jax.experimental import pallas as pl
from jax.experimental.pallas import tpu as pltpu
from jax.experimental.pallas import tpu_sc as plsc

N = 10000
R = 8
D = 128
E = 320000
EE = 2 * E

NC = 2
NS = 16
NW = NC * NS
PER_W = EE // NW
CHUNK = 80
NCHUNK = PER_W // CHUNK

NP = 10240
ROWS_PER_SUB = NP // NS
DROWS = NP // 128

RBLK = 1000
EBLK = 1024


def _ht_body(h_ref, w_ref, out_ref):
    out_ref[...] = jnp.dot(h_ref[...], w_ref[0],
                           preferred_element_type=jnp.float32)


def _ht_table(h, w_rel):
    nb = N // RBLK
    return pl.pallas_call(
        _ht_body,
        grid=(nb, 2 * R),
        in_specs=[
            pl.BlockSpec((RBLK, D), lambda i, r: (i, 0)),
            pl.BlockSpec((1, D, D), lambda i, r: (r, 0, 0)),
        ],
        out_specs=pl.BlockSpec((RBLK, D), lambda i, r: (r * nb + i, 0)),
        out_shape=jax.ShapeDtypeStruct((2 * R * N, D), jnp.float32),
    )(h, w_rel)


def _sc_body(ht_hbm, gidx_hbm, dst_hbm, zrow_hbm,
             agg_out, deg_out,
             gbuf0, gbuf1, dbuf0, dbuf1, qbuf, rows0, rows1, ones_rows,
             agg_sh, deg_sh, si0, si1, sg0, sg1):
    cid = lax.axis_index("c")
    sid = lax.axis_index("s")
    wid = sid * NC + cid
    gbuf = (gbuf0, gbuf1)
    dbuf = (dbuf0, dbuf1)
    rows = (rows0, rows1)
    si = (si0, si1)
    sg = (sg0, sg1)
    sl = pl.ds(sid * ROWS_PER_SUB, ROWS_PER_SUB)
    pltpu.sync_copy(zrow_hbm, agg_sh.at[sl])
    pltpu.sync_copy(zrow_hbm.at[pl.ds(0, CHUNK)], ones_rows)

    @pl.when(sid == 0)
    def _():
        pltpu.sync_copy(zrow_hbm.at[pl.ds(0, DROWS)], deg_sh)

    plsc.subcore_barrier()

    base = wid * PER_W

    def _off(cc):
        return pl.multiple_of(base + cc * CHUNK, 8)

    def _issue_idx(cc, b):
        off = _off(cc)
        pltpu.async_copy(gidx_hbm.at[pl.ds(off, CHUNK)], gbuf[b], si[b])
        pltpu.async_copy(dst_hbm.at[pl.ds(off, CHUNK)], dbuf[b], si[b])

    def _wait_idx(b):
        pltpu.make_async_copy(gidx_hbm.at[pl.ds(0, CHUNK)], gbuf[b],
                              si[b]).wait()
        pltpu.make_async_copy(dst_hbm.at[pl.ds(0, CHUNK)], dbuf[b],
                              si[b]).wait()

    _issue_idx(0, 0)
    _issue_idx(1, 1)
    _wait_idx(0)
    pltpu.async_copy(ht_hbm.at[gbuf[0]], rows[0], sg[0])

    ones16 = jnp.ones((16,), jnp.float32)
    zero16 = jnp.zeros((16,), jnp.float32)

    def _consume(b):
        pltpu.sync_copy(rows[b], agg_sh.at[dbuf[b]], add=True)
        for k in range(CHUNK // 16):
            dv = dbuf[b][pl.ds(k * 16, 16)]
            qbuf[pl.ds(k * 16, 16)] = lax.shift_right_logical(dv, 7)
            evec = lax.iota(jnp.int32, 16) + (k * 16)
            gvec = lax.bitwise_and(dv, 127)
            plsc.store_scatter(ones_rows, [evec, gvec], ones16)
        pltpu.sync_copy(ones_rows, deg_sh.at[qbuf], add=True)
        for k in range(CHUNK // 16):
            dv = dbuf[b][pl.ds(k * 16, 16)]
            evec = lax.iota(jnp.int32, 16) + (k * 16)
            gvec = lax.bitwise_and(dv, 127)
            plsc.store_scatter(ones_rows, [evec, gvec], zero16)

    @pl.loop(0, NCHUNK // 2)
    def _(i):
        for b in (0, 1):
            cc = i * 2 + b
            o = 1 - b

            @pl.when(cc + 1 < NCHUNK)
            def _():
                _wait_idx(o)
                pltpu.async_copy(ht_hbm.at[gbuf[o]], rows[o], sg[o])

            pltpu.make_async_copy(ht_hbm.at[gbuf[b]], rows[b], sg[b]).wait()
            _consume(b)

            @pl.when(cc + 2 < NCHUNK)
            def _():
                _issue_idx(cc + 2, b)

    plsc.subcore_barrier()
    pltpu.sync_copy(agg_sh.at[sl], agg_out.at[cid, sl])

    @pl.when(sid == 0)
    def _():
        pltpu.sync_copy(deg_sh, deg_out.at[cid])


def _sc_aggregate(ht, gidx, dst, zrow):
    mesh = plsc.VectorSubcoreMesh(core_axis_name="c", subcore_axis_name="s",
                                  num_cores=NC, num_subcores=NS)
    cp = pltpu.CompilerParams()
    if "needs_layout_passes" in pltpu.CompilerParams.__dataclass_fields__:
        cp = dataclasses.replace(cp, needs_layout_passes=False)
    k = pl.kernel(
        _sc_body,
        out_type=[
            jax.ShapeDtypeStruct((NC, NP, D), jnp.float32),
            jax.ShapeDtypeStruct((NC, DROWS, D), jnp.float32),
        ],
        mesh=mesh,
        scratch_types=[
            pltpu.VMEM((CHUNK,), jnp.int32),
            pltpu.VMEM((CHUNK,), jnp.int32),
            pltpu.VMEM((CHUNK,), jnp.int32),
            pltpu.VMEM((CHUNK,), jnp.int32),
            pltpu.VMEM((CHUNK,), jnp.int32),
            pltpu.VMEM((CHUNK, D), jnp.float32),
            pltpu.VMEM((CHUNK, D), jnp.float32),
            pltpu.VMEM((CHUNK, D), jnp.float32),
            pltpu.VMEM_SHARED((NP, D), jnp.float32),
            pltpu.VMEM_SHARED((DROWS, D), jnp.float32),
            pltpu.SemaphoreType.DMA,
            pltpu.SemaphoreType.DMA,
            pltpu.SemaphoreType.DMA,
            pltpu.SemaphoreType.DMA,
        ],
        compiler_params=cp,
    )
    return k(ht, gidx, dst, zrow)


def _epi_body(agg_ref, scale_ref, h_ref, ws_ref, out_ref):
    agg = agg_ref[0] + agg_ref[1]
    hs = jnp.dot(h_ref[...], ws_ref[...], preferred_element_type=jnp.float32)
    out_ref[...] = jnp.maximum(agg * scale_ref[...] + hs, 0.0)


def _epilogue(agg, scale_b, h_pad, w_self):
    nb = NP // EBLK
    return pl.pallas_call(
        _epi_body,
        grid=(nb,),
        in_specs=[
            pl.BlockSpec((NC, EBLK, D), lambda i: (0, i, 0)),
            pl.BlockSpec((EBLK, D), lambda i: (i, 0)),
            pl.BlockSpec((EBLK, D), lambda i: (i, 0)),
            pl.BlockSpec((D, D), lambda i: (0, 0)),
        ],
        out_specs=pl.BlockSpec((EBLK, D), lambda i: (i, 0)),
        out_shape=jax.ShapeDtypeStruct((NP, D), jnp.float32),
    )(agg, scale_b, h_pad, w_self)


def kernel(edges, entity_embed, W_rel, W_self):
    src = edges[:, 0]
    rel = edges[:, 1]
    dst = edges[:, 2]
    gidx = jnp.concatenate([rel * N + src, (rel + R) * N + dst])
    dall = jnp.concatenate([dst, src])

    ht = _ht_table(entity_embed, W_rel)
    zrow = jnp.zeros((ROWS_PER_SUB, D), jnp.float32)
    aggs, degs = _sc_aggregate(ht, gidx, dall, zrow)

    deg = (degs[0] + degs[1]).reshape(NP)
    scale = 1.0 / jnp.clip(deg, 1.0, None)
    scale_b = jnp.broadcast_to(scale[:, None], (NP, D))
    h_pad = jnp.pad(entity_embed, ((0, NP - N), (0, 0)))
    out = _epilogue(aggs, scale_b, h_pad, W_self)
    return out[:N]


# speedup vs baseline: 7.0225x; 1.1448x over previous
import dataclasses

import jax
import jax.numpy as jnp
from jax import lax
from jax.experimental import pallas as pl
from jax.experimental.pallas import tpu as pltpu
from jax.experimental.pallas import tpu_sc as plsc

N = 10000
R = 8
D = 128
E = 320000
EE = 2 * E

NC = 2
NS = 16
NW = NC * NS
PER_W = EE // NW
CHUNK = 80
NCHUNK = PER_W // CHUNK

NP = 10240
ROWS_PER_SUB = NP // NS
DROWS = NP // 16

RBLK = 1000
EBLK = 1024


def _ht_body(h_ref, w_ref, out_ref):
    out_ref[...] = jnp.dot(h_ref[...], w_ref[0],
                           preferred_element_type=jnp.float32)


def _ht_table(h, w_rel):
    nb = N // RBLK
    return pl.pallas_call(
        _ht_body,
        grid=(nb, 2 * R),
        in_specs=[
            pl.BlockSpec((RBLK, D), lambda i, r: (i, 0)),
            pl.BlockSpec((1, D, D), lambda i, r: (r, 0, 0)),
        ],
        out_specs=pl.BlockSpec((RBLK, D), lambda i, r: (r * nb + i, 0)),
        out_shape=jax.ShapeDtypeStruct((2 * R * N, D), jnp.float32),
    )(h, w_rel)


def _sc_body(ht_hbm, gidx_hbm, dst_hbm, zrow_hbm,
             agg_out, deg_out,
             gbuf0, gbuf1, dbuf0, dbuf1, qbuf, rows0, rows1, ones_rows,
             agg_sh, deg_sh, si0, si1, sg0, sg1):
    cid = lax.axis_index("c")
    sid = lax.axis_index("s")
    wid = sid * NC + cid
    gbuf = (gbuf0, gbuf1)
    dbuf = (dbuf0, dbuf1)
    rows = (rows0, rows1)
    si = (si0, si1)
    sg = (sg0, sg1)
    sl = pl.ds(sid * ROWS_PER_SUB, ROWS_PER_SUB)
    pltpu.sync_copy(zrow_hbm, agg_sh.at[sl])

    zero16 = jnp.zeros((16,), jnp.float32)

    @pl.loop(0, CHUNK)
    def _(i):
        ones_rows[i, :] = zero16

    pltpu.sync_copy(ones_rows.at[pl.ds(0, DROWS // NS)],
                    deg_sh.at[pl.ds(sid * (DROWS // NS), DROWS // NS)])

    plsc.subcore_barrier()

    base = wid * PER_W

    def _off(cc):
        return pl.multiple_of(base + cc * CHUNK, 8)

    def _issue_idx(cc, b):
        off = _off(cc)
        pltpu.async_copy(gidx_hbm.at[pl.ds(off, CHUNK)], gbuf[b], si[b])
        pltpu.async_copy(dst_hbm.at[pl.ds(off, CHUNK)], dbuf[b], si[b])

    def _wait_idx(b):
        pltpu.make_async_copy(gidx_hbm.at[pl.ds(0, CHUNK)], gbuf[b],
                              si[b]).wait()
        pltpu.make_async_copy(dst_hbm.at[pl.ds(0, CHUNK)], dbuf[b],
                              si[b]).wait()

    _issue_idx(0, 0)
    _issue_idx(1, 1)
    _wait_idx(0)
    pltpu.async_copy(ht_hbm.at[gbuf[0]], rows[0], sg[0])

    ones16 = jnp.ones((16,), jnp.float32)

    def _consume(b):
        pltpu.sync_copy(rows[b], agg_sh.at[dbuf[b]], add=True)
        for k in range(CHUNK // 16):
            dv = dbuf[b][pl.ds(k * 16, 16)]
            qbuf[pl.ds(k * 16, 16)] = lax.shift_right_logical(dv, 4)
            evec = lax.iota(jnp.int32, 16) + (k * 16)
            gvec = lax.bitwise_and(dv, 15)
            plsc.store_scatter(ones_rows, [evec, gvec], ones16)
        pltpu.sync_copy(ones_rows, deg_sh.at[qbuf], add=True)
        for k in range(CHUNK // 16):
            dv = dbuf[b][pl.ds(k * 16, 16)]
            evec = lax.iota(jnp.int32, 16) + (k * 16)
            gvec = lax.bitwise_and(dv, 15)
            plsc.store_scatter(ones_rows, [evec, gvec], zero16)

    @pl.loop(0, NCHUNK // 2)
    def _(i):
        for b in (0, 1):
            cc = i * 2 + b
            o = 1 - b

            @pl.when(cc + 1 < NCHUNK)
            def _():
                _wait_idx(o)
                pltpu.async_copy(ht_hbm.at[gbuf[o]], rows[o], sg[o])

            pltpu.make_async_copy(ht_hbm.at[gbuf[b]], rows[b], sg[b]).wait()
            _consume(b)

            @pl.when(cc + 2 < NCHUNK)
            def _():
                _issue_idx(cc + 2, b)

    plsc.subcore_barrier()
    pltpu.sync_copy(agg_sh.at[sl], agg_out.at[cid, sl])

    @pl.when(sid == 0)
    def _():
        pltpu.sync_copy(deg_sh, deg_out.at[cid])


def _sc_aggregate(ht, gidx, dst, zrow):
    mesh = plsc.VectorSubcoreMesh(core_axis_name="c", subcore_axis_name="s",
                                  num_cores=NC, num_subcores=NS)
    cp = pltpu.CompilerParams()
    if "needs_layout_passes" in pltpu.CompilerParams.__dataclass_fields__:
        cp = dataclasses.replace(cp, needs_layout_passes=False)
    k = pl.kernel(
        _sc_body,
        out_type=[
            jax.ShapeDtypeStruct((NC, NP, D), jnp.float32),
            jax.ShapeDtypeStruct((NC, DROWS, 16), jnp.float32),
        ],
        mesh=mesh,
        scratch_types=[
            pltpu.VMEM((CHUNK,), jnp.int32),
            pltpu.VMEM((CHUNK,), jnp.int32),
            pltpu.VMEM((CHUNK,), jnp.int32),
            pltpu.VMEM((CHUNK,), jnp.int32),
            pltpu.VMEM((CHUNK,), jnp.int32),
            pltpu.VMEM((CHUNK, D), jnp.float32),
            pltpu.VMEM((CHUNK, D), jnp.float32),
            pltpu.VMEM((CHUNK, 16), jnp.float32),
            pltpu.VMEM_SHARED((NP, D), jnp.float32),
            pltpu.VMEM_SHARED((DROWS, 16), jnp.float32),
            pltpu.SemaphoreType.DMA,
            pltpu.SemaphoreType.DMA,
            pltpu.SemaphoreType.DMA,
            pltpu.SemaphoreType.DMA,
        ],
        compiler_params=cp,
    )
    return k(ht, gidx, dst, zrow)


def _epi_body(agg_ref, scale_ref, h_ref, ws_ref, out_ref):
    agg = agg_ref[0] + agg_ref[1]
    hs = jnp.dot(h_ref[...], ws_ref[...], preferred_element_type=jnp.float32)
    out_ref[...] = jnp.maximum(agg * scale_ref[...] + hs, 0.0)


def _epilogue(agg, scale_b, h_pad, w_self):
    nb = NP // EBLK
    return pl.pallas_call(
        _epi_body,
        grid=(nb,),
        in_specs=[
            pl.BlockSpec((NC, EBLK, D), lambda i: (0, i, 0)),
            pl.BlockSpec((EBLK, D), lambda i: (i, 0)),
            pl.BlockSpec((EBLK, D), lambda i: (i, 0)),
            pl.BlockSpec((D, D), lambda i: (0, 0)),
        ],
        out_specs=pl.BlockSpec((EBLK, D), lambda i: (i, 0)),
        out_shape=jax.ShapeDtypeStruct((NP, D), jnp.float32),
    )(agg, scale_b, h_pad, w_self)


def kernel(edges, entity_embed, W_rel, W_self):
    src = edges[:, 0]
    rel = edges[:, 1]
    dst = edges[:, 2]
    gidx = jnp.concatenate([rel * N + src, (rel + R) * N + dst])
    dall = jnp.concatenate([dst, src])

    ht = _ht_table(entity_embed, W_rel)
    zrow = jnp.zeros((ROWS_PER_SUB, D), jnp.float32)
    aggs, degs = _sc_aggregate(ht, gidx, dall, zrow)

    deg = (degs[0] + degs[1]).reshape(NP)
    scale = 1.0 / jnp.clip(deg, 1.0, None)
    scale_b = jnp.broadcast_to(scale[:, None], (NP, D))
    h_pad = jnp.pad(entity_embed, ((0, NP - N), (0, 0)))
    out = _epilogue(aggs, scale_b, h_pad, W_self)
    return out[:N]
